# baseline (speedup 1.0000x reference)
import jax
import jax.numpy as jnp
from jax import lax
from jax.experimental import pallas as pl
from jax.experimental.pallas import tpu as pltpu
from jax.experimental.pallas import tpu_sc as plsc

BATCH = 16384
N_FIELDS = 26
NUM_CORES = 2
NUM_SUBCORES = 16
LANES = 16
NW = NUM_CORES * NUM_SUBCORES
BPW = BATCH // NW
IPW = BPW * N_FIELDS
K = 4
BPC = BPW // K
IPC = BPC * N_FIELDS
SUBCHUNKS = BPC // LANES


def _body(idx_hbm, table_hbm, out_hbm, idx_v, rows_v, out_v, *sems):
    wid = lax.axis_index("s") * NUM_CORES + lax.axis_index("c")
    ibase = wid * IPW

    pltpu.sync_copy(idx_hbm.at[pl.ds(ibase, IPC)], idx_v.at[pl.ds(0, IPC)])
    copies = [None] * K
    copies[0] = pltpu.async_copy(
        table_hbm.at[idx_v.at[pl.ds(0, IPC)]], rows_v.at[pl.ds(0, IPC)], sems[0]
    )
    pltpu.sync_copy(
        idx_hbm.at[pl.ds(ibase + IPC, IPW - IPC)],
        idx_v.at[pl.ds(IPC, IPW - IPC)],
    )
    for c in range(1, K):
        copies[c] = pltpu.async_copy(
            table_hbm.at[idx_v.at[pl.ds(c * IPC, IPC)]],
            rows_v.at[pl.ds(c * IPC, IPC)],
            sems[c],
        )

    for c in range(K):
        copies[c].wait()
        cbase = c * IPC

        def step(s, carry, cbase=cbase, c=c):
            off = cbase + s * LANES
            acc = rows_v[pl.ds(off, LANES)]
            for f in range(1, N_FIELDS):
                acc = acc + rows_v[pl.ds(off + f * BPC, LANES)]
            out_v[pl.ds(c * BPC + s * LANES, LANES)] = acc
            return carry

        lax.fori_loop(0, SUBCHUNKS, step, 0)

    pltpu.sync_copy(out_v, out_hbm.at[pl.ds(wid * BPW, BPW)])


@jax.jit
def _run(idx_flat, table):
    mesh = plsc.VectorSubcoreMesh(core_axis_name="c", subcore_axis_name="s")
    k = pl.kernel(
        _body,
        mesh=mesh,
        out_type=jax.ShapeDtypeStruct((BATCH,), jnp.float32),
        scratch_types=[
            pltpu.VMEM((IPW,), jnp.int32),
            pltpu.VMEM((IPW,), jnp.float32),
            pltpu.VMEM((BPW,), jnp.float32),
        ]
        + [pltpu.SemaphoreType.DMA] * K,
    )
    return k(idx_flat, table)


def kernel(inputs, w):
    idx_flat = (
        inputs.astype(jnp.int32)
        .reshape(NW, K, BPC, N_FIELDS)
        .transpose(0, 1, 3, 2)
        .reshape(-1)
    )
    idx_flat = (idx_flat % 8) * 125000 + idx_flat // 8
    table = w.reshape(125000, 8).T.reshape(-1)
    out = _run(idx_flat, table)
    return out.reshape(BATCH, 1)


# speedup vs baseline: 3.6538x; 3.6538x over previous
import jax
import jax.numpy as jnp
from jax import lax
from jax.experimental import pallas as pl
from jax.experimental.pallas import tpu as pltpu
from jax.experimental.pallas import tpu_sc as plsc

BATCH = 16384
N_FIELDS = 26
NUM_CORES = 2
NUM_SUBCORES = 16
LANES = 16
NW = NUM_CORES * NUM_SUBCORES
BPW = BATCH // NW
IPW = BPW * N_FIELDS
K = 4
BPC = BPW // K
IPC = BPC * N_FIELDS
SUBCHUNKS = BPC // LANES


def _body(idx_hbm, table_hbm, out_hbm, idx_v, rows_v, out_v, *sems):
    wid = lax.axis_index("s") * NUM_CORES + lax.axis_index("c")
    ibase = wid * IPW

    pltpu.sync_copy(idx_hbm.at[pl.ds(ibase, IPC)], idx_v.at[pl.ds(0, IPC)])
    copies = [None] * K
    copies[0] = pltpu.async_copy(
        table_hbm.at[idx_v.at[pl.ds(0, IPC)]], rows_v.at[pl.ds(0, IPC)], sems[0]
    )
    pltpu.sync_copy(
        idx_hbm.at[pl.ds(ibase + IPC, IPW - IPC)],
        idx_v.at[pl.ds(IPC, IPW - IPC)],
    )
    for c in range(1, K):
        copies[c] = pltpu.async_copy(
            table_hbm.at[idx_v.at[pl.ds(c * IPC, IPC)]],
            rows_v.at[pl.ds(c * IPC, IPC)],
            sems[c],
        )

    for c in range(K):
        copies[c].wait()
        cbase = c * IPC

        def step(s, carry, cbase=cbase, c=c):
            off = cbase + s * LANES
            acc = rows_v[pl.ds(off, LANES)]
            for f in range(1, N_FIELDS):
                acc = acc + rows_v[pl.ds(off + f * BPC, LANES)]
            out_v[pl.ds(c * BPC + s * LANES, LANES)] = acc
            return carry

        lax.fori_loop(0, SUBCHUNKS, step, 0)

    pltpu.sync_copy(out_v, out_hbm.at[pl.ds(wid * BPW, BPW)])


@jax.jit
def _run(idx_flat, table):
    mesh = plsc.VectorSubcoreMesh(core_axis_name="c", subcore_axis_name="s")
    k = pl.kernel(
        _body,
        mesh=mesh,
        out_type=jax.ShapeDtypeStruct((BATCH,), jnp.float32),
        scratch_types=[
            pltpu.VMEM((IPW,), jnp.int32),
            pltpu.VMEM((IPW,), jnp.float32),
            pltpu.VMEM((BPW,), jnp.float32),
        ]
        + [pltpu.SemaphoreType.DMA] * K,
    )
    return k(idx_flat, table)


def kernel(inputs, w):
    idx_flat = (
        inputs.astype(jnp.int32)
        .reshape(NW, K, BPC, N_FIELDS)
        .transpose(0, 1, 3, 2)
        .reshape(-1)
    )
    table = jnp.pad(w, ((0, 448), (0, 0))).reshape(-1)
    out = _run(idx_flat, table)
    return out.reshape(BATCH, 1)
